# baseline (speedup 1.0000x reference)
import functools

import jax
import jax.numpy as jnp
from jax import lax
from jax.experimental import pallas as pl
from jax.experimental.pallas import tpu as pltpu
from jax.experimental.pallas import tpu_sc as plsc

N = 10000
E = 320000
D = 128
NCLS = 40
T = 5000
T_PAD = 5120
PAD_IDX = 10008
M_ACC = 10016

NC = 2
NS = 16
EPT = E // (NC * NS)
EPT_H = E // NS
WRT = 632
WRT_L = N - (NS - 1) * WRT

_mesh = plsc.VectorSubcoreMesh(core_axis_name="c", subcore_axis_name="s")


@functools.partial(
    pl.kernel, mesh=_mesh,
    out_type=[jax.ShapeDtypeStruct((N,), jnp.float32),
              jax.ShapeDtypeStruct((N,), jnp.float32),
              jax.ShapeDtypeStruct((N,), jnp.float32)],
    scratch_types=[pltpu.VMEM_SHARED((M_ACC,), jnp.float32),
                   pltpu.VMEM_SHARED((M_ACC,), jnp.float32),
                   pltpu.VMEM((M_ACC,), jnp.float32),
                   pltpu.VMEM((128,), jnp.int32),
                   pltpu.VMEM((128,), jnp.int32),
                   pltpu.VMEM((128,), jnp.int32),
                   pltpu.VMEM((16,), jnp.int32),
                   pltpu.VMEM((64,), jnp.int32),
                   pltpu.VMEM((128,), jnp.float32)]
                  + [pltpu.SemaphoreType.DMA] * 6)
def _hist(dst1d, train1d, zeros1, deg0_out, deg1_out, m_out,
          acc_sh, m_sh, vbuf, ix0, ix1, ix2, idx_t16, idx_t64, ones_v,
          hI0, hI1, hI2, hS0, hS1, hS2):
    idx = (ix0, ix1, ix2)
    semI = (hI0, hI1, hI2)
    semS = (hS0, hS1, hS2)
    cid = lax.axis_index("c")
    sid = lax.axis_index("s")
    w = sid * NC + cid
    ebase = pl.multiple_of(w * EPT, 8)
    NCH = EPT // 128
    for k in range(8):
        ones_v[pl.ds(k * 16, 16)] = jnp.ones((16,), jnp.float32)

    @pl.when(sid == 0)
    def _():
        pltpu.sync_copy(zeros1, vbuf)
        pltpu.sync_copy(vbuf, acc_sh)

    @pl.when((cid == 1) & (sid == 1))
    def _():
        pltpu.sync_copy(zeros1, vbuf)
        pltpu.sync_copy(vbuf, m_sh)

    plsc.subcore_barrier()

    def _start_idx(j, b):
        off = pl.multiple_of(ebase + j * 128, 8)
        pltpu.async_copy(dst1d.at[pl.ds(off, 128)], idx[b], semI[b])

    for b in range(3):
        _start_idx(b, b)

    def body(c, _):
        for b in range(3):
            pltpu.make_async_copy(dst1d.at[pl.ds(0, 128)], idx[b],
                                  semI[b]).wait()
            pltpu.async_copy(ones_v, acc_sh.at[idx[b]], semS[b], add=True)
        for b in range(3):
            jn = 3 * c + b + 3
            pltpu.make_async_copy(ones_v, acc_sh.at[idx[b]],
                                  semS[b]).wait()

            @pl.when(jn < NCH)
            def _():
                _start_idx(jn, b)
        return 0
    lax.fori_loop(0, NCH // 3, body, 0)

    toff = pl.multiple_of(ebase + NCH * 128, 8)
    pltpu.sync_copy(dst1d.at[pl.ds(toff, 16)], idx_t16)
    pltpu.sync_copy(ones_v.at[pl.ds(0, 16)], acc_sh.at[idx_t16], add=True)

    @pl.when(cid == 1)
    def _():
        tbase = pl.multiple_of(sid * (T_PAD // NS), 8)
        for j in range(2):
            pltpu.sync_copy(train1d.at[pl.ds(tbase + j * 128, 128)], ix0)
            pltpu.sync_copy(ones_v, m_sh.at[ix0], add=True)
        pltpu.sync_copy(train1d.at[pl.ds(tbase + 256, 64)], idx_t64)
        pltpu.sync_copy(ones_v.at[pl.ds(0, 64)], m_sh.at[idx_t64], add=True)

    plsc.subcore_barrier()

    @pl.when((cid == 0) & (sid == 0))
    def _():
        pltpu.sync_copy(acc_sh.at[pl.ds(0, N)], vbuf.at[pl.ds(0, N)])
        pltpu.sync_copy(vbuf.at[pl.ds(0, N)], deg0_out)

    @pl.when((cid == 1) & (sid == 0))
    def _():
        pltpu.sync_copy(acc_sh.at[pl.ds(0, N)], vbuf.at[pl.ds(0, N)])
        pltpu.sync_copy(vbuf.at[pl.ds(0, N)], deg1_out)

    @pl.when((cid == 1) & (sid == 1))
    def _():
        pltpu.sync_copy(m_sh.at[pl.ds(0, N)], vbuf.at[pl.ds(0, N)])
        pltpu.sync_copy(vbuf.at[pl.ds(0, N)], m_out)


@functools.partial(
    pl.kernel, mesh=_mesh,
    out_type=[jax.ShapeDtypeStruct((N, D), jnp.float32),
              jax.ShapeDtypeStruct((N, D), jnp.float32)],
    scratch_types=[pltpu.VMEM_SHARED((N, D), jnp.float32)]
                  + [pltpu.VMEM((40,), jnp.int32)] * 8
                  + [pltpu.VMEM((40,), jnp.int32)] * 8
                  + [pltpu.VMEM((40, D), jnp.float32)] * 8
                  + [pltpu.SemaphoreType.DMA] * 24)
def _agg(z_hbm, src1d, dst1d, zrows, out0, out1, acc_sh, *bufs):
    NB = 4
    NSL = 2 * NB
    sidx = tuple(bufs[0:8])
    didx = tuple(bufs[8:16])
    rows = tuple(bufs[16:24])
    semI = tuple(bufs[24:32])
    semG = tuple(bufs[32:40])
    semS = tuple(bufs[40:48])
    cid = lax.axis_index("c")
    sid = lax.axis_index("s")
    w = sid * NC + cid
    ebase = pl.multiple_of(w * EPT, 8)
    rbase = pl.multiple_of(sid * WRT, 8)
    CH = 40
    NCH = EPT // CH

    pltpu.sync_copy(zrows, rows[0])

    def _zsweep(total):
        done = 0
        while done < total:
            c = min(CH, total - done)
            pltpu.sync_copy(rows[0].at[pl.ds(0, c)],
                            acc_sh.at[pl.ds(rbase + done, c)])
            done += c

    @pl.when(sid < NS - 1)
    def _():
        _zsweep(WRT)

    @pl.when(sid == NS - 1)
    def _():
        _zsweep(WRT_L)

    plsc.subcore_barrier()

    def _sIs(j, s):
        off = pl.multiple_of(ebase + j * CH, 8)
        pltpu.async_copy(src1d.at[pl.ds(off, CH)], sidx[s], semI[s])

    def _sId(j, s):
        off = pl.multiple_of(ebase + j * CH, 8)
        pltpu.async_copy(dst1d.at[pl.ds(off, CH)], didx[s], semI[s])

    def _wI(s):
        pltpu.make_async_copy(src1d.at[pl.ds(0, CH)], sidx[s],
                              semI[s]).wait()
        pltpu.make_async_copy(dst1d.at[pl.ds(0, CH)], didx[s],
                              semI[s]).wait()

    def _wS(s):
        pltpu.make_async_copy(rows[s], acc_sh.at[didx[s]], semS[s]).wait()

    for p in range(2):
        for b in range(NB):
            _sIs(NB * p + b, 2 * b + p)
            _sId(NB * p + b, 2 * b + p)

    NIT = NCH // NSL

    def body(c, _):
        for p in range(2):
            for b in range(NB):
                s = 2 * b + p
                j = NSL * c + NB * p + b

                @pl.when(c > 0)
                def _():
                    _wS(s)
                    _sId(j, s)
                _wI(s)
                pltpu.async_copy(z_hbm.at[sidx[s]], rows[s], semG[s])
            for b in range(NB):
                s = 2 * b + p
                j = NSL * c + NB * p + b
                pltpu.make_async_copy(z_hbm.at[pl.ds(0, CH)], rows[s],
                                      semG[s]).wait()
                pltpu.async_copy(rows[s], acc_sh.at[didx[s]], semS[s],
                                 add=True)
                jn = j + NSL

                @pl.when(jn < NIT * NSL)
                def _():
                    _sIs(jn, s)
        return 0
    lax.fori_loop(0, NIT, body, 0)
    for s in range(NSL):
        _wS(s)

    for j in (248, 249):
        off = pl.multiple_of(ebase + j * CH, 8)
        pltpu.sync_copy(src1d.at[pl.ds(off, CH)], sidx[0])
        pltpu.sync_copy(dst1d.at[pl.ds(off, CH)], didx[0])
        pltpu.sync_copy(z_hbm.at[sidx[0]], rows[0])
        pltpu.sync_copy(rows[0], acc_sh.at[didx[0]], add=True)

    plsc.subcore_barrier()

    def _writeout(out_ref, total):
        done = 0
        while done < total:
            c = min(CH, total - done)
            pltpu.sync_copy(acc_sh.at[pl.ds(rbase + done, c)],
                            rows[0].at[pl.ds(0, c)])
            pltpu.sync_copy(rows[0].at[pl.ds(0, c)],
                            out_ref.at[pl.ds(rbase + done, c)])
            done += c

    @pl.when((cid == 0) & (sid < NS - 1))
    def _():
        _writeout(out0, WRT)

    @pl.when((cid == 0) & (sid == NS - 1))
    def _():
        _writeout(out0, WRT_L)

    @pl.when((cid == 1) & (sid < NS - 1))
    def _():
        _writeout(out1, WRT)

    @pl.when((cid == 1) & (sid == NS - 1))
    def _():
        _writeout(out1, WRT_L)


_BLK = 1000
_GRID = N // _BLK


def _a_body(f_ref, w_ref, d0_ref, d1_ref, z_ref):
    dis = lax.rsqrt(d0_ref[...] + d1_ref[...] + 1.0)
    z_ref[...] = jnp.dot(f_ref[...], w_ref[...],
                         preferred_element_type=jnp.float32) * dis


def _mid_body(a0_ref, a1_ref, z_ref, d0_ref, d1_ref, b_ref, w_ref, out_ref,
              *, relu):
    dis = lax.rsqrt(d0_ref[...] + d1_ref[...] + 1.0)
    h = (a0_ref[...] + a1_ref[...] + z_ref[...]) * dis + b_ref[...]
    if relu:
        h = jnp.maximum(h, 0.0)
    out_ref[...] = jnp.dot(h, w_ref[...],
                           preferred_element_type=jnp.float32) * dis


def _d_body(a0_ref, a1_ref, z_ref, d0_ref, d1_ref, b_ref, w3_ref, b3_ref,
            lab_ref, m_ref, pred_ref, hn_ref, loss_ref):
    i = pl.program_id(0)
    dis = lax.rsqrt(d0_ref[...] + d1_ref[...] + 1.0)
    h = (a0_ref[...] + a1_ref[...] + z_ref[...]) * dis + b_ref[...]
    h = jnp.maximum(h, 0.0)
    nrm = jnp.sqrt(jnp.sum(h * h, axis=1, keepdims=True))
    hn = h / jnp.maximum(nrm, 1e-12)
    hn_ref[...] = hn
    logits = jnp.dot(hn, w3_ref[...],
                     preferred_element_type=jnp.float32) + b3_ref[...]
    mx = jnp.max(logits, axis=1, keepdims=True)
    lse = mx + jnp.log(jnp.sum(jnp.exp(logits - mx), axis=1, keepdims=True))
    pred = logits - lse
    pred_ref[...] = pred
    cls = lax.broadcasted_iota(jnp.int32, (_BLK, NCLS), 1)
    sel = jnp.where(cls == lab_ref[...], pred, 0.0)
    part = jnp.sum(sel * m_ref[...])
    tot = jnp.where(i == 0, part, loss_ref[0, 0] + part)
    loss_ref[0, 0] = jnp.where(i == _GRID - 1, -tot / T, tot)


def _row_spec():
    return pl.BlockSpec((_BLK, D), lambda i: (i, 0))


def _col_spec():
    return pl.BlockSpec((_BLK, 1), lambda i: (i, 0))


def _full_spec(shape):
    return pl.BlockSpec(shape, lambda i: tuple(0 for _ in shape))


_tc_a = pl.pallas_call(
    _a_body, grid=(_GRID,),
    in_specs=[_row_spec(), _full_spec((D, D)), _col_spec(), _col_spec()],
    out_specs=_row_spec(),
    out_shape=jax.ShapeDtypeStruct((N, D), jnp.float32))

_tc_mid = pl.pallas_call(
    functools.partial(_mid_body, relu=False), grid=(_GRID,),
    in_specs=[_row_spec(), _row_spec(), _row_spec(), _col_spec(),
              _col_spec(), _full_spec((1, D)), _full_spec((D, D))],
    out_specs=_row_spec(),
    out_shape=jax.ShapeDtypeStruct((N, D), jnp.float32))

_tc_mid_relu = pl.pallas_call(
    functools.partial(_mid_body, relu=True), grid=(_GRID,),
    in_specs=[_row_spec(), _row_spec(), _row_spec(), _col_spec(),
              _col_spec(), _full_spec((1, D)), _full_spec((D, D))],
    out_specs=_row_spec(),
    out_shape=jax.ShapeDtypeStruct((N, D), jnp.float32))

_tc_d = pl.pallas_call(
    _d_body, grid=(_GRID,),
    in_specs=[_row_spec(), _row_spec(), _row_spec(), _col_spec(),
              _col_spec(), _full_spec((1, D)), _full_spec((D, NCLS)),
              _full_spec((1, NCLS)), _col_spec(), _col_spec()],
    out_specs=[pl.BlockSpec((_BLK, NCLS), lambda i: (i, 0)),
               _row_spec(),
               pl.BlockSpec((1, 1), lambda i: (0, 0),
                            memory_space=pltpu.SMEM)],
    out_shape=[jax.ShapeDtypeStruct((N, NCLS), jnp.float32),
               jax.ShapeDtypeStruct((N, D), jnp.float32),
               jax.ShapeDtypeStruct((1, 1), jnp.float32)])


def kernel(feats, edge_list, train_idx, labels,
           gc1_W, gc1_b, gc2_W0, gc2_b0, gc2_W1, gc2_b1, gc3_W, gc3_b):
    src1d = edge_list[0]
    dst1d = edge_list[1]
    train1d = jnp.pad(train_idx, (0, T_PAD - T), constant_values=PAD_IDX)
    zeros1 = jnp.zeros((M_ACC,), jnp.float32)
    zrows = jnp.zeros((40, D), jnp.float32)
    deg0, deg1, m = _hist(dst1d, train1d, zeros1)
    dg0 = deg0.reshape(N, 1)
    dg1 = deg1.reshape(N, 1)
    m2d = m.reshape(N, 1)
    lab2d = labels.reshape(N, 1)
    b1 = gc1_b.reshape(1, D)
    b20 = gc2_b0.reshape(1, D)
    b21 = gc2_b1.reshape(1, D)
    b3 = gc3_b.reshape(1, NCLS)

    z1 = _tc_a(feats, gc1_W, dg0, dg1)
    a10, a11 = _agg(z1, src1d, dst1d, zrows)
    z2 = _tc_mid(a10, a11, z1, dg0, dg1, b1, gc2_W0)
    a20, a21 = _agg(z2, src1d, dst1d, zrows)
    z3 = _tc_mid_relu(a20, a21, z2, dg0, dg1, b20, gc2_W1)
    a30, a31 = _agg(z3, src1d, dst1d, zrows)
    pred, hn, lsum = _tc_d(a30, a31, z3, dg0, dg1, b21, gc3_W, b3,
                           lab2d, m2d)
    return (lsum[0, 0], pred, hn)


# speedup vs baseline: 1.3944x; 1.3944x over previous
import functools

import jax
import jax.numpy as jnp
from jax import lax
from jax.experimental import pallas as pl
from jax.experimental.pallas import tpu as pltpu
from jax.experimental.pallas import tpu_sc as plsc

N = 10000
E = 320000
D = 128
NCLS = 40
T = 5000
T_PAD = 5120
PAD_IDX = 10008
M_ACC = 10016

NC = 2
NS = 16
EPT = E // (NC * NS)
EPT_H = E // NS
WRT = 632
WRT_L = N - (NS - 1) * WRT

_mesh = plsc.VectorSubcoreMesh(core_axis_name="c", subcore_axis_name="s")


@functools.partial(
    pl.kernel, mesh=_mesh,
    out_type=[jax.ShapeDtypeStruct((N,), jnp.float32),
              jax.ShapeDtypeStruct((N,), jnp.float32),
              jax.ShapeDtypeStruct((N,), jnp.float32)],
    scratch_types=[pltpu.VMEM_SHARED((M_ACC,), jnp.float32),
                   pltpu.VMEM_SHARED((M_ACC,), jnp.float32),
                   pltpu.VMEM((M_ACC,), jnp.float32),
                   pltpu.VMEM((128,), jnp.int32),
                   pltpu.VMEM((128,), jnp.int32),
                   pltpu.VMEM((128,), jnp.int32),
                   pltpu.VMEM((16,), jnp.int32),
                   pltpu.VMEM((64,), jnp.int32),
                   pltpu.VMEM((128,), jnp.float32)]
                  + [pltpu.SemaphoreType.DMA] * 6)
def _hist(dst1d, train1d, zeros1, deg0_out, deg1_out, m_out,
          acc_sh, m_sh, vbuf, ix0, ix1, ix2, idx_t16, idx_t64, ones_v,
          hI0, hI1, hI2, hS0, hS1, hS2):
    idx = (ix0, ix1, ix2)
    semI = (hI0, hI1, hI2)
    semS = (hS0, hS1, hS2)
    cid = lax.axis_index("c")
    sid = lax.axis_index("s")
    w = sid * NC + cid
    ebase = pl.multiple_of(w * EPT, 8)
    NCH = EPT // 128
    for k in range(8):
        ones_v[pl.ds(k * 16, 16)] = jnp.ones((16,), jnp.float32)

    @pl.when(sid == 0)
    def _():
        pltpu.sync_copy(zeros1, vbuf)
        pltpu.sync_copy(vbuf, acc_sh)

    @pl.when((cid == 1) & (sid == 1))
    def _():
        pltpu.sync_copy(zeros1, vbuf)
        pltpu.sync_copy(vbuf, m_sh)

    plsc.subcore_barrier()

    def _start_idx(j, b):
        off = pl.multiple_of(ebase + j * 128, 8)
        pltpu.async_copy(dst1d.at[pl.ds(off, 128)], idx[b], semI[b])

    for b in range(3):
        _start_idx(b, b)

    def body(c, _):
        for b in range(3):
            pltpu.make_async_copy(dst1d.at[pl.ds(0, 128)], idx[b],
                                  semI[b]).wait()
            pltpu.async_copy(ones_v, acc_sh.at[idx[b]], semS[b], add=True)
        for b in range(3):
            jn = 3 * c + b + 3
            pltpu.make_async_copy(ones_v, acc_sh.at[idx[b]],
                                  semS[b]).wait()

            @pl.when(jn < NCH)
            def _():
                _start_idx(jn, b)
        return 0
    lax.fori_loop(0, NCH // 3, body, 0)

    toff = pl.multiple_of(ebase + NCH * 128, 8)
    pltpu.sync_copy(dst1d.at[pl.ds(toff, 16)], idx_t16)
    pltpu.sync_copy(ones_v.at[pl.ds(0, 16)], acc_sh.at[idx_t16], add=True)

    @pl.when(cid == 1)
    def _():
        tbase = pl.multiple_of(sid * (T_PAD // NS), 8)
        for j in range(2):
            pltpu.sync_copy(train1d.at[pl.ds(tbase + j * 128, 128)], ix0)
            pltpu.sync_copy(ones_v, m_sh.at[ix0], add=True)
        pltpu.sync_copy(train1d.at[pl.ds(tbase + 256, 64)], idx_t64)
        pltpu.sync_copy(ones_v.at[pl.ds(0, 64)], m_sh.at[idx_t64], add=True)

    plsc.subcore_barrier()

    @pl.when((cid == 0) & (sid == 0))
    def _():
        pltpu.sync_copy(acc_sh.at[pl.ds(0, N)], vbuf.at[pl.ds(0, N)])
        pltpu.sync_copy(vbuf.at[pl.ds(0, N)], deg0_out)

    @pl.when((cid == 1) & (sid == 0))
    def _():
        pltpu.sync_copy(acc_sh.at[pl.ds(0, N)], vbuf.at[pl.ds(0, N)])
        pltpu.sync_copy(vbuf.at[pl.ds(0, N)], deg1_out)

    @pl.when((cid == 1) & (sid == 1))
    def _():
        pltpu.sync_copy(m_sh.at[pl.ds(0, N)], vbuf.at[pl.ds(0, N)])
        pltpu.sync_copy(vbuf.at[pl.ds(0, N)], m_out)


@functools.partial(
    pl.kernel, mesh=_mesh,
    out_type=[jax.ShapeDtypeStruct((N, D), jnp.float32),
              jax.ShapeDtypeStruct((N, D), jnp.float32)],
    scratch_types=[pltpu.VMEM_SHARED((N, D), jnp.float32)]
                  + [pltpu.VMEM((80,), jnp.int32)] * 4
                  + [pltpu.VMEM((80,), jnp.int32)] * 4
                  + [pltpu.VMEM((80, D), jnp.float32)] * 4
                  + [pltpu.SemaphoreType.DMA] * 12)
def _agg(z_hbm, src1d, dst1d, zrows, out0, out1, acc_sh, *bufs):
    NSL = 4
    sidx = tuple(bufs[0:4])
    didx = tuple(bufs[4:8])
    rows = tuple(bufs[8:12])
    semI = tuple(bufs[12:16])
    semG = tuple(bufs[16:20])
    semS = tuple(bufs[20:24])
    cid = lax.axis_index("c")
    sid = lax.axis_index("s")
    w = sid * NC + cid
    ebase = pl.multiple_of(w * EPT, 8)
    rbase = pl.multiple_of(sid * WRT, 8)
    CH = 80
    NCH = EPT // CH

    pltpu.sync_copy(zrows, rows[0])

    def _zsweep(total):
        done = 0
        while done < total:
            c = min(CH, total - done)
            pltpu.sync_copy(rows[0].at[pl.ds(0, c)],
                            acc_sh.at[pl.ds(rbase + done, c)])
            done += c

    @pl.when(sid < NS - 1)
    def _():
        _zsweep(WRT)

    @pl.when(sid == NS - 1)
    def _():
        _zsweep(WRT_L)

    plsc.subcore_barrier()

    def _sIs(j, s):
        off = pl.multiple_of(ebase + j * CH, 8)
        pltpu.async_copy(src1d.at[pl.ds(off, CH)], sidx[s], semI[s])

    def _sId(j, s):
        off = pl.multiple_of(ebase + j * CH, 8)
        pltpu.async_copy(dst1d.at[pl.ds(off, CH)], didx[s], semI[s])

    def _wI(s):
        pltpu.make_async_copy(src1d.at[pl.ds(0, CH)], sidx[s],
                              semI[s]).wait()
        pltpu.make_async_copy(dst1d.at[pl.ds(0, CH)], didx[s],
                              semI[s]).wait()

    def _wS(s):
        pltpu.make_async_copy(rows[s], acc_sh.at[didx[s]], semS[s]).wait()

    for s in range(NSL):
        _sIs(s, s)
        _sId(s, s)

    NIT = NCH // NSL

    def body(c, _):
        for s in range(NSL):
            j = NSL * c + s

            @pl.when(c > 0)
            def _():
                _wS(s)
                _sId(j, s)
            _wI(s)
            pltpu.async_copy(z_hbm.at[sidx[s]], rows[s], semG[s])
        for s in range(NSL):
            j = NSL * c + s
            pltpu.make_async_copy(z_hbm.at[pl.ds(0, CH)], rows[s],
                                  semG[s]).wait()
            pltpu.async_copy(rows[s], acc_sh.at[didx[s]], semS[s],
                             add=True)
            jn = j + NSL

            @pl.when(jn < NIT * NSL)
            def _():
                _sIs(jn, s)
        return 0
    lax.fori_loop(0, NIT, body, 0)
    for s in range(NSL):
        _wS(s)

    for j in (124,):
        off = pl.multiple_of(ebase + j * CH, 8)
        pltpu.sync_copy(src1d.at[pl.ds(off, CH)], sidx[0])
        pltpu.sync_copy(dst1d.at[pl.ds(off, CH)], didx[0])
        pltpu.sync_copy(z_hbm.at[sidx[0]], rows[0])
        pltpu.sync_copy(rows[0], acc_sh.at[didx[0]], add=True)

    plsc.subcore_barrier()

    def _writeout(out_ref, total):
        done = 0
        while done < total:
            c = min(CH, total - done)
            pltpu.sync_copy(acc_sh.at[pl.ds(rbase + done, c)],
                            rows[0].at[pl.ds(0, c)])
            pltpu.sync_copy(rows[0].at[pl.ds(0, c)],
                            out_ref.at[pl.ds(rbase + done, c)])
            done += c

    @pl.when((cid == 0) & (sid < NS - 1))
    def _():
        _writeout(out0, WRT)

    @pl.when((cid == 0) & (sid == NS - 1))
    def _():
        _writeout(out0, WRT_L)

    @pl.when((cid == 1) & (sid < NS - 1))
    def _():
        _writeout(out1, WRT)

    @pl.when((cid == 1) & (sid == NS - 1))
    def _():
        _writeout(out1, WRT_L)


_BLK = 1000
_GRID = N // _BLK


def _a_body(f_ref, w_ref, d0_ref, d1_ref, z_ref):
    dis = lax.rsqrt(d0_ref[...] + d1_ref[...] + 1.0)
    z_ref[...] = jnp.dot(f_ref[...], w_ref[...],
                         preferred_element_type=jnp.float32) * dis


def _mid_body(a0_ref, a1_ref, z_ref, d0_ref, d1_ref, b_ref, w_ref, out_ref,
              *, relu):
    dis = lax.rsqrt(d0_ref[...] + d1_ref[...] + 1.0)
    h = (a0_ref[...] + a1_ref[...] + z_ref[...]) * dis + b_ref[...]
    if relu:
        h = jnp.maximum(h, 0.0)
    out_ref[...] = jnp.dot(h, w_ref[...],
                           preferred_element_type=jnp.float32) * dis


def _d_body(a0_ref, a1_ref, z_ref, d0_ref, d1_ref, b_ref, w3_ref, b3_ref,
            lab_ref, m_ref, pred_ref, hn_ref, loss_ref):
    i = pl.program_id(0)
    dis = lax.rsqrt(d0_ref[...] + d1_ref[...] + 1.0)
    h = (a0_ref[...] + a1_ref[...] + z_ref[...]) * dis + b_ref[...]
    h = jnp.maximum(h, 0.0)
    nrm = jnp.sqrt(jnp.sum(h * h, axis=1, keepdims=True))
    hn = h / jnp.maximum(nrm, 1e-12)
    hn_ref[...] = hn
    logits = jnp.dot(hn, w3_ref[...],
                     preferred_element_type=jnp.float32) + b3_ref[...]
    mx = jnp.max(logits, axis=1, keepdims=True)
    lse = mx + jnp.log(jnp.sum(jnp.exp(logits - mx), axis=1, keepdims=True))
    pred = logits - lse
    pred_ref[...] = pred
    cls = lax.broadcasted_iota(jnp.int32, (_BLK, NCLS), 1)
    sel = jnp.where(cls == lab_ref[...], pred, 0.0)
    part = jnp.sum(sel * m_ref[...])
    tot = jnp.where(i == 0, part, loss_ref[0, 0] + part)
    loss_ref[0, 0] = jnp.where(i == _GRID - 1, -tot / T, tot)


def _row_spec():
    return pl.BlockSpec((_BLK, D), lambda i: (i, 0))


def _col_spec():
    return pl.BlockSpec((_BLK, 1), lambda i: (i, 0))


def _full_spec(shape):
    return pl.BlockSpec(shape, lambda i: tuple(0 for _ in shape))


_tc_a = pl.pallas_call(
    _a_body, grid=(_GRID,),
    in_specs=[_row_spec(), _full_spec((D, D)), _col_spec(), _col_spec()],
    out_specs=_row_spec(),
    out_shape=jax.ShapeDtypeStruct((N, D), jnp.float32))

_tc_mid = pl.pallas_call(
    functools.partial(_mid_body, relu=False), grid=(_GRID,),
    in_specs=[_row_spec(), _row_spec(), _row_spec(), _col_spec(),
              _col_spec(), _full_spec((1, D)), _full_spec((D, D))],
    out_specs=_row_spec(),
    out_shape=jax.ShapeDtypeStruct((N, D), jnp.float32))

_tc_mid_relu = pl.pallas_call(
    functools.partial(_mid_body, relu=True), grid=(_GRID,),
    in_specs=[_row_spec(), _row_spec(), _row_spec(), _col_spec(),
              _col_spec(), _full_spec((1, D)), _full_spec((D, D))],
    out_specs=_row_spec(),
    out_shape=jax.ShapeDtypeStruct((N, D), jnp.float32))

_tc_d = pl.pallas_call(
    _d_body, grid=(_GRID,),
    in_specs=[_row_spec(), _row_spec(), _row_spec(), _col_spec(),
              _col_spec(), _full_spec((1, D)), _full_spec((D, NCLS)),
              _full_spec((1, NCLS)), _col_spec(), _col_spec()],
    out_specs=[pl.BlockSpec((_BLK, NCLS), lambda i: (i, 0)),
               _row_spec(),
               pl.BlockSpec((1, 1), lambda i: (0, 0),
                            memory_space=pltpu.SMEM)],
    out_shape=[jax.ShapeDtypeStruct((N, NCLS), jnp.float32),
               jax.ShapeDtypeStruct((N, D), jnp.float32),
               jax.ShapeDtypeStruct((1, 1), jnp.float32)])


def kernel(feats, edge_list, train_idx, labels,
           gc1_W, gc1_b, gc2_W0, gc2_b0, gc2_W1, gc2_b1, gc3_W, gc3_b):
    src1d = edge_list[0]
    dst1d = edge_list[1]
    train1d = jnp.pad(train_idx, (0, T_PAD - T), constant_values=PAD_IDX)
    zeros1 = jnp.zeros((M_ACC,), jnp.float32)
    zrows = jnp.zeros((80, D), jnp.float32)
    deg0, deg1, m = _hist(dst1d, train1d, zeros1)
    dg0 = deg0.reshape(N, 1)
    dg1 = deg1.reshape(N, 1)
    m2d = m.reshape(N, 1)
    lab2d = labels.reshape(N, 1)
    b1 = gc1_b.reshape(1, D)
    b20 = gc2_b0.reshape(1, D)
    b21 = gc2_b1.reshape(1, D)
    b3 = gc3_b.reshape(1, NCLS)

    z1 = _tc_a(feats, gc1_W, dg0, dg1)
    a10, a11 = _agg(z1, src1d, dst1d, zrows)
    z2 = _tc_mid(a10, a11, z1, dg0, dg1, b1, gc2_W0)
    a20, a21 = _agg(z2, src1d, dst1d, zrows)
    z3 = _tc_mid_relu(a20, a21, z2, dg0, dg1, b20, gc2_W1)
    a30, a31 = _agg(z3, src1d, dst1d, zrows)
    pred, hn, lsum = _tc_d(a30, a31, z3, dg0, dg1, b21, gc3_W, b3,
                           lab2d, m2d)
    return (lsum[0, 0], pred, hn)


# speedup vs baseline: 1.4160x; 1.0155x over previous
import functools

import jax
import jax.numpy as jnp
from jax import lax
from jax.experimental import pallas as pl
from jax.experimental.pallas import tpu as pltpu
from jax.experimental.pallas import tpu_sc as plsc

N = 10000
E = 320000
D = 128
NCLS = 40
T = 5000
T_PAD = 5120
PAD_IDX = 10008
M_ACC = 10016

NC = 2
NS = 16
EPT = E // (NC * NS)
EPT_H = E // NS
WRT = 632
WRT_L = N - (NS - 1) * WRT

_mesh = plsc.VectorSubcoreMesh(core_axis_name="c", subcore_axis_name="s")


@functools.partial(
    pl.kernel, mesh=_mesh,
    out_type=[jax.ShapeDtypeStruct((N,), jnp.float32),
              jax.ShapeDtypeStruct((N,), jnp.float32),
              jax.ShapeDtypeStruct((N,), jnp.float32)],
    scratch_types=[pltpu.VMEM_SHARED((M_ACC,), jnp.float32),
                   pltpu.VMEM_SHARED((M_ACC,), jnp.float32),
                   pltpu.VMEM((M_ACC,), jnp.float32),
                   pltpu.VMEM((128,), jnp.int32),
                   pltpu.VMEM((128,), jnp.int32),
                   pltpu.VMEM((128,), jnp.int32),
                   pltpu.VMEM((16,), jnp.int32),
                   pltpu.VMEM((64,), jnp.int32),
                   pltpu.VMEM((128,), jnp.float32)]
                  + [pltpu.SemaphoreType.DMA] * 6)
def _hist(dst1d, train1d, zeros1, deg0_out, deg1_out, m_out,
          acc_sh, m_sh, vbuf, ix0, ix1, ix2, idx_t16, idx_t64, ones_v,
          hI0, hI1, hI2, hS0, hS1, hS2):
    idx = (ix0, ix1, ix2)
    semI = (hI0, hI1, hI2)
    semS = (hS0, hS1, hS2)
    cid = lax.axis_index("c")
    sid = lax.axis_index("s")
    w = sid * NC + cid
    ebase = pl.multiple_of(w * EPT, 8)
    NCH = EPT // 128
    for k in range(8):
        ones_v[pl.ds(k * 16, 16)] = jnp.ones((16,), jnp.float32)

    @pl.when(sid == 0)
    def _():
        pltpu.sync_copy(zeros1, vbuf)
        pltpu.sync_copy(vbuf, acc_sh)

    @pl.when((cid == 1) & (sid == 1))
    def _():
        pltpu.sync_copy(zeros1, vbuf)
        pltpu.sync_copy(vbuf, m_sh)

    plsc.subcore_barrier()

    def _start_idx(j, b):
        off = pl.multiple_of(ebase + j * 128, 8)
        pltpu.async_copy(dst1d.at[pl.ds(off, 128)], idx[b], semI[b])

    for b in range(3):
        _start_idx(b, b)

    def body(c, _):
        for b in range(3):
            pltpu.make_async_copy(dst1d.at[pl.ds(0, 128)], idx[b],
                                  semI[b]).wait()
            pltpu.async_copy(ones_v, acc_sh.at[idx[b]], semS[b], add=True)
        for b in range(3):
            jn = 3 * c + b + 3
            pltpu.make_async_copy(ones_v, acc_sh.at[idx[b]],
                                  semS[b]).wait()

            @pl.when(jn < NCH)
            def _():
                _start_idx(jn, b)
        return 0
    lax.fori_loop(0, NCH // 3, body, 0)

    toff = pl.multiple_of(ebase + NCH * 128, 8)
    pltpu.sync_copy(dst1d.at[pl.ds(toff, 16)], idx_t16)
    pltpu.sync_copy(ones_v.at[pl.ds(0, 16)], acc_sh.at[idx_t16], add=True)

    @pl.when(cid == 1)
    def _():
        tbase = pl.multiple_of(sid * (T_PAD // NS), 8)
        for j in range(2):
            pltpu.sync_copy(train1d.at[pl.ds(tbase + j * 128, 128)], ix0)
            pltpu.sync_copy(ones_v, m_sh.at[ix0], add=True)
        pltpu.sync_copy(train1d.at[pl.ds(tbase + 256, 64)], idx_t64)
        pltpu.sync_copy(ones_v.at[pl.ds(0, 64)], m_sh.at[idx_t64], add=True)

    plsc.subcore_barrier()

    @pl.when((cid == 0) & (sid == 0))
    def _():
        pltpu.sync_copy(acc_sh.at[pl.ds(0, N)], vbuf.at[pl.ds(0, N)])
        pltpu.sync_copy(vbuf.at[pl.ds(0, N)], deg0_out)

    @pl.when((cid == 1) & (sid == 0))
    def _():
        pltpu.sync_copy(acc_sh.at[pl.ds(0, N)], vbuf.at[pl.ds(0, N)])
        pltpu.sync_copy(vbuf.at[pl.ds(0, N)], deg1_out)

    @pl.when((cid == 1) & (sid == 1))
    def _():
        pltpu.sync_copy(m_sh.at[pl.ds(0, N)], vbuf.at[pl.ds(0, N)])
        pltpu.sync_copy(vbuf.at[pl.ds(0, N)], m_out)


@functools.partial(
    pl.kernel, mesh=_mesh,
    out_type=[jax.ShapeDtypeStruct((N, D), jnp.float32),
              jax.ShapeDtypeStruct((N, D), jnp.float32)],
    scratch_types=[pltpu.VMEM_SHARED((N, D), jnp.float32)]
                  + [pltpu.VMEM((64,), jnp.int32)] * 6
                  + [pltpu.VMEM((64,), jnp.int32)] * 6
                  + [pltpu.VMEM((16,), jnp.int32)]
                  + [pltpu.VMEM((64, D), jnp.float32)] * 6
                  + [pltpu.SemaphoreType.DMA] * 24)
def _agg(z_hbm, src1d, dst1d, zrows, out0, out1, acc_sh, *bufs):
    sidx = tuple(bufs[0:6])
    didx = tuple(bufs[6:12])
    didx_t = bufs[12]
    rows = tuple(bufs[13:19])
    semIs = tuple(bufs[19:25])
    semId = tuple(bufs[25:31])
    semG = tuple(bufs[31:37])
    semS = tuple(bufs[37:43])
    cid = lax.axis_index("c")
    sid = lax.axis_index("s")
    w = sid * NC + cid
    ebase = pl.multiple_of(w * EPT, 8)
    rbase = pl.multiple_of(sid * WRT, 8)
    CH = 64
    NCH = EPT // CH

    pltpu.sync_copy(zrows, rows[0])

    def _zsweep(total):
        done = 0
        while done < total:
            c = min(CH, total - done)
            pltpu.sync_copy(rows[0].at[pl.ds(0, c)],
                            acc_sh.at[pl.ds(rbase + done, c)])
            done += c

    @pl.when(sid < NS - 1)
    def _():
        _zsweep(WRT)

    @pl.when(sid == NS - 1)
    def _():
        _zsweep(WRT_L)

    plsc.subcore_barrier()

    def _sIs(j, s):
        pltpu.async_copy(
            src1d.at[pl.ds(pl.multiple_of(ebase + j * CH, 8), CH)],
            sidx[s], semIs[s])

    def _sId(j, s):
        pltpu.async_copy(
            dst1d.at[pl.ds(pl.multiple_of(ebase + j * CH, 8), CH)],
            didx[s], semId[s])

    def _wS(s):
        pltpu.make_async_copy(rows[s], acc_sh.at[didx[s]], semS[s]).wait()

    for p in range(2):
        for b in range(3):
            _sIs(3 * p + b, 2 * b + p)

    def body(c, _):
        for p in range(2):
            for b in range(3):
                s = 2 * b + p
                j = 6 * c + 3 * p + b

                @pl.when(c > 0)
                def _():
                    _wS(s)
                _sId(j, s)
                pltpu.make_async_copy(src1d.at[pl.ds(0, CH)], sidx[s],
                                      semIs[s]).wait()
                pltpu.async_copy(z_hbm.at[sidx[s]], rows[s], semG[s])
            for b in range(3):
                s = 2 * b + p
                j = 6 * c + 3 * p + b
                pltpu.make_async_copy(z_hbm.at[pl.ds(0, CH)], rows[s],
                                      semG[s]).wait()
                pltpu.make_async_copy(dst1d.at[pl.ds(0, CH)], didx[s],
                                      semId[s]).wait()
                pltpu.async_copy(rows[s], acc_sh.at[didx[s]], semS[s],
                                 add=True)
                jn = j + 6

                @pl.when(jn < NCH)
                def _():
                    _sIs(jn, s)
        return 0
    lax.fori_loop(0, NCH // 6, body, 0)
    for s in range(6):
        _wS(s)

    toff = NCH * CH
    pltpu.sync_copy(src1d.at[pl.ds(pl.multiple_of(ebase + toff, 8), 16)],
                    sidx[0].at[pl.ds(0, 16)])
    pltpu.sync_copy(dst1d.at[pl.ds(pl.multiple_of(ebase + toff, 8), 16)],
                    didx_t)
    pltpu.sync_copy(z_hbm.at[sidx[0].at[pl.ds(0, 16)]],
                    rows[1].at[pl.ds(0, 16)])
    pltpu.sync_copy(rows[1].at[pl.ds(0, 16)], acc_sh.at[didx_t], add=True)

    plsc.subcore_barrier()

    def _writeout(out_ref, total):
        done = 0
        while done < total:
            c = min(CH, total - done)
            pltpu.sync_copy(acc_sh.at[pl.ds(rbase + done, c)],
                            rows[0].at[pl.ds(0, c)])
            pltpu.sync_copy(rows[0].at[pl.ds(0, c)],
                            out_ref.at[pl.ds(rbase + done, c)])
            done += c

    @pl.when((cid == 0) & (sid < NS - 1))
    def _():
        _writeout(out0, WRT)

    @pl.when((cid == 0) & (sid == NS - 1))
    def _():
        _writeout(out0, WRT_L)

    @pl.when((cid == 1) & (sid < NS - 1))
    def _():
        _writeout(out1, WRT)

    @pl.when((cid == 1) & (sid == NS - 1))
    def _():
        _writeout(out1, WRT_L)


_BLK = 1000
_GRID = N // _BLK


def _a_body(f_ref, w_ref, d0_ref, d1_ref, z_ref):
    dis = lax.rsqrt(d0_ref[...] + d1_ref[...] + 1.0)
    z_ref[...] = jnp.dot(f_ref[...], w_ref[...],
                         preferred_element_type=jnp.float32) * dis


def _mid_body(a0_ref, a1_ref, z_ref, d0_ref, d1_ref, b_ref, w_ref, out_ref,
              *, relu):
    dis = lax.rsqrt(d0_ref[...] + d1_ref[...] + 1.0)
    h = (a0_ref[...] + a1_ref[...] + z_ref[...]) * dis + b_ref[...]
    if relu:
        h = jnp.maximum(h, 0.0)
    out_ref[...] = jnp.dot(h, w_ref[...],
                           preferred_element_type=jnp.float32) * dis


def _d_body(a0_ref, a1_ref, z_ref, d0_ref, d1_ref, b_ref, w3_ref, b3_ref,
            lab_ref, m_ref, pred_ref, hn_ref, loss_ref):
    i = pl.program_id(0)
    dis = lax.rsqrt(d0_ref[...] + d1_ref[...] + 1.0)
    h = (a0_ref[...] + a1_ref[...] + z_ref[...]) * dis + b_ref[...]
    h = jnp.maximum(h, 0.0)
    nrm = jnp.sqrt(jnp.sum(h * h, axis=1, keepdims=True))
    hn = h / jnp.maximum(nrm, 1e-12)
    hn_ref[...] = hn
    logits = jnp.dot(hn, w3_ref[...],
                     preferred_element_type=jnp.float32) + b3_ref[...]
    mx = jnp.max(logits, axis=1, keepdims=True)
    lse = mx + jnp.log(jnp.sum(jnp.exp(logits - mx), axis=1, keepdims=True))
    pred = logits - lse
    pred_ref[...] = pred
    cls = lax.broadcasted_iota(jnp.int32, (_BLK, NCLS), 1)
    sel = jnp.where(cls == lab_ref[...], pred, 0.0)
    part = jnp.sum(sel * m_ref[...])
    tot = jnp.where(i == 0, part, loss_ref[0, 0] + part)
    loss_ref[0, 0] = jnp.where(i == _GRID - 1, -tot / T, tot)


def _row_spec():
    return pl.BlockSpec((_BLK, D), lambda i: (i, 0))


def _col_spec():
    return pl.BlockSpec((_BLK, 1), lambda i: (i, 0))


def _full_spec(shape):
    return pl.BlockSpec(shape, lambda i: tuple(0 for _ in shape))


_tc_a = pl.pallas_call(
    _a_body, grid=(_GRID,),
    in_specs=[_row_spec(), _full_spec((D, D)), _col_spec(), _col_spec()],
    out_specs=_row_spec(),
    out_shape=jax.ShapeDtypeStruct((N, D), jnp.float32))

_tc_mid = pl.pallas_call(
    functools.partial(_mid_body, relu=False), grid=(_GRID,),
    in_specs=[_row_spec(), _row_spec(), _row_spec(), _col_spec(),
              _col_spec(), _full_spec((1, D)), _full_spec((D, D))],
    out_specs=_row_spec(),
    out_shape=jax.ShapeDtypeStruct((N, D), jnp.float32))

_tc_mid_relu = pl.pallas_call(
    functools.partial(_mid_body, relu=True), grid=(_GRID,),
    in_specs=[_row_spec(), _row_spec(), _row_spec(), _col_spec(),
              _col_spec(), _full_spec((1, D)), _full_spec((D, D))],
    out_specs=_row_spec(),
    out_shape=jax.ShapeDtypeStruct((N, D), jnp.float32))

_tc_d = pl.pallas_call(
    _d_body, grid=(_GRID,),
    in_specs=[_row_spec(), _row_spec(), _row_spec(), _col_spec(),
              _col_spec(), _full_spec((1, D)), _full_spec((D, NCLS)),
              _full_spec((1, NCLS)), _col_spec(), _col_spec()],
    out_specs=[pl.BlockSpec((_BLK, NCLS), lambda i: (i, 0)),
               _row_spec(),
               pl.BlockSpec((1, 1), lambda i: (0, 0),
                            memory_space=pltpu.SMEM)],
    out_shape=[jax.ShapeDtypeStruct((N, NCLS), jnp.float32),
               jax.ShapeDtypeStruct((N, D), jnp.float32),
               jax.ShapeDtypeStruct((1, 1), jnp.float32)])


def kernel(feats, edge_list, train_idx, labels,
           gc1_W, gc1_b, gc2_W0, gc2_b0, gc2_W1, gc2_b1, gc3_W, gc3_b):
    src1d = edge_list[0]
    dst1d = edge_list[1]
    train1d = jnp.pad(train_idx, (0, T_PAD - T), constant_values=PAD_IDX)
    zeros1 = jnp.zeros((M_ACC,), jnp.float32)
    zrows = jnp.zeros((64, D), jnp.float32)
    deg0, deg1, m = _hist(dst1d, train1d, zeros1)
    dg0 = deg0.reshape(N, 1)
    dg1 = deg1.reshape(N, 1)
    m2d = m.reshape(N, 1)
    lab2d = labels.reshape(N, 1)
    b1 = gc1_b.reshape(1, D)
    b20 = gc2_b0.reshape(1, D)
    b21 = gc2_b1.reshape(1, D)
    b3 = gc3_b.reshape(1, NCLS)

    z1 = _tc_a(feats, gc1_W, dg0, dg1)
    a10, a11 = _agg(z1, src1d, dst1d, zrows)
    z2 = _tc_mid(a10, a11, z1, dg0, dg1, b1, gc2_W0)
    a20, a21 = _agg(z2, src1d, dst1d, zrows)
    z3 = _tc_mid_relu(a20, a21, z2, dg0, dg1, b20, gc2_W1)
    a30, a31 = _agg(z3, src1d, dst1d, zrows)
    pred, hn, lsum = _tc_d(a30, a31, z3, dg0, dg1, b21, gc3_W, b3,
                           lab2d, m2d)
    return (lsum[0, 0], pred, hn)
